# baseline (speedup 1.0000x reference)
import functools

import jax
import jax.numpy as jnp
from jax import lax
from jax.experimental import pallas as pl
from jax.experimental.pallas import tpu as pltpu
from jax.experimental.pallas import tpu_sc as plsc

DIM = 128
HALF = DIM // 2
MAXP = 8192
NC = 2
NS = 16
NW = NC * NS
TOTAL = 4 * 4096
BPW = TOTAL // NW
CH = 64
NCH = BPW // CH

_mesh = plsc.VectorSubcoreMesh(core_axis_name="c", subcore_axis_name="s")


@functools.partial(
    pl.kernel,
    mesh=_mesh,
    compiler_params=pltpu.CompilerParams(
        use_tc_tiling_on_sc=False, skip_device_barrier=True
    ),
    out_type=(
        jax.ShapeDtypeStruct((TOTAL, 2, HALF), jnp.float32),
        jax.ShapeDtypeStruct((TOTAL, 2, HALF), jnp.float32),
    ),
    scratch_types=[
        pltpu.VMEM((BPW,), jnp.int32),
        pltpu.VMEM((NCH, CH, HALF), jnp.float32),
        pltpu.VMEM((NCH, CH, HALF), jnp.float32),
        pltpu.SemaphoreType.DMA,
        pltpu.SemaphoreType.DMA,
        pltpu.SemaphoreType.DMA,
        pltpu.SemaphoreType.DMA,
    ],
)
def _gather_kernel(cos_hbm, sin_hbm, pos_hbm, cos_out, sin_out,
                   idx_v, cbuf, sbuf, gcsem, gssem, wcsem, wssem):
    wid = lax.axis_index("s") * NC + lax.axis_index("c")
    base = wid * BPW
    pltpu.sync_copy(pos_hbm.at[pl.ds(base, BPW)], idx_v)
    gc, gs, writes = {}, {}, []
    for c in range(NCH):
        idx = idx_v.at[pl.ds(c * CH, CH)]
        gc[c] = pltpu.async_copy(cos_hbm.at[idx], cbuf.at[c], gcsem)
        gs[c] = pltpu.async_copy(sin_hbm.at[idx], sbuf.at[c], gssem)
    for c in range(NCH):
        out = pl.ds(base + c * CH, CH)
        gc[c].wait()
        writes.append(pltpu.async_copy(cbuf.at[c], cos_out.at[out, 0], wcsem))
        writes.append(pltpu.async_copy(cbuf.at[c], cos_out.at[out, 1], wcsem))
        gs[c].wait()
        writes.append(pltpu.async_copy(sbuf.at[c], sin_out.at[out, 0], wssem))
        writes.append(pltpu.async_copy(sbuf.at[c], sin_out.at[out, 1], wssem))
    for w in writes:
        w.wait()


def kernel(cos_cached, sin_cached, position_ids):
    b, s = position_ids.shape
    cos_h = cos_cached.reshape(2 * MAXP, HALF)
    sin_h = sin_cached.reshape(2 * MAXP, HALF)
    pos = (position_ids.astype(jnp.int32) * 2).reshape(TOTAL)
    cos_full, sin_full = _gather_kernel(cos_h, sin_h, pos)
    return (cos_full.reshape(b, s, DIM), sin_full.reshape(b, s, DIM))


# speedup vs baseline: 1.0265x; 1.0265x over previous
import functools

import jax
import jax.numpy as jnp
from jax import lax
from jax.experimental import pallas as pl
from jax.experimental.pallas import tpu as pltpu
from jax.experimental.pallas import tpu_sc as plsc

DIM = 128
HALF = DIM // 2
MAXP = 8192
NC = 2
NS = 16
NW = NC * NS
TOTAL = 4 * 4096
BPW = TOTAL // NW
CH = 128
NCH = BPW // CH

_mesh = plsc.VectorSubcoreMesh(core_axis_name="c", subcore_axis_name="s")


@functools.partial(
    pl.kernel,
    mesh=_mesh,
    compiler_params=pltpu.CompilerParams(
        use_tc_tiling_on_sc=False, skip_device_barrier=True
    ),
    out_type=(
        jax.ShapeDtypeStruct((TOTAL, 2, HALF), jnp.float32),
        jax.ShapeDtypeStruct((TOTAL, 2, HALF), jnp.float32),
    ),
    scratch_types=[
        pltpu.VMEM((BPW,), jnp.int32),
        pltpu.VMEM((NCH, CH, HALF), jnp.float32),
        pltpu.VMEM((NCH, CH, HALF), jnp.float32),
        pltpu.SemaphoreType.DMA,
        pltpu.SemaphoreType.DMA,
        pltpu.SemaphoreType.DMA,
        pltpu.SemaphoreType.DMA,
    ],
)
def _gather_kernel(cos_hbm, sin_hbm, pos_hbm, cos_out, sin_out,
                   idx_v, cbuf, sbuf, gcsem, gssem, wcsem, wssem):
    wid = lax.axis_index("s") * NC + lax.axis_index("c")
    base = wid * BPW
    pltpu.sync_copy(pos_hbm.at[pl.ds(base, BPW)], idx_v)
    gc, gs, writes = {}, {}, []
    for c in range(NCH):
        idx = idx_v.at[pl.ds(c * CH, CH)]
        gc[c] = pltpu.async_copy(cos_hbm.at[idx], cbuf.at[c], gcsem)
        gs[c] = pltpu.async_copy(sin_hbm.at[idx], sbuf.at[c], gssem)
    for c in range(NCH):
        out = pl.ds(base + c * CH, CH)
        gc[c].wait()
        writes.append(pltpu.async_copy(cbuf.at[c], cos_out.at[out, 0], wcsem))
        writes.append(pltpu.async_copy(cbuf.at[c], cos_out.at[out, 1], wcsem))
        gs[c].wait()
        writes.append(pltpu.async_copy(sbuf.at[c], sin_out.at[out, 0], wssem))
        writes.append(pltpu.async_copy(sbuf.at[c], sin_out.at[out, 1], wssem))
    for w in writes:
        w.wait()


def kernel(cos_cached, sin_cached, position_ids):
    b, s = position_ids.shape
    cos_h = cos_cached.reshape(2 * MAXP, HALF)
    sin_h = sin_cached.reshape(2 * MAXP, HALF)
    pos = (position_ids.astype(jnp.int32) * 2).reshape(TOTAL)
    cos_full, sin_full = _gather_kernel(cos_h, sin_h, pos)
    return (cos_full.reshape(b, s, DIM), sin_full.reshape(b, s, DIM))


# speedup vs baseline: 1.0278x; 1.0013x over previous
import functools

import jax
import jax.numpy as jnp
from jax import lax
from jax.experimental import pallas as pl
from jax.experimental.pallas import tpu as pltpu
from jax.experimental.pallas import tpu_sc as plsc

DIM = 128
HALF = DIM // 2
MAXP = 8192
NC = 2
NS = 16
NW = NC * NS
TOTAL = 4 * 4096
BPW = TOTAL // NW
CH = 128
NCH = BPW // CH

_mesh = plsc.VectorSubcoreMesh(core_axis_name="c", subcore_axis_name="s")


@functools.partial(
    pl.kernel,
    mesh=_mesh,
    compiler_params=pltpu.CompilerParams(use_tc_tiling_on_sc=False),
    out_type=(
        jax.ShapeDtypeStruct((TOTAL, 2, HALF), jnp.float32),
        jax.ShapeDtypeStruct((TOTAL, 2, HALF), jnp.float32),
    ),
    scratch_types=[
        pltpu.VMEM((BPW,), jnp.int32),
        pltpu.VMEM((NCH, CH, HALF), jnp.float32),
        pltpu.VMEM((NCH, CH, HALF), jnp.float32),
        pltpu.SemaphoreType.DMA,
        pltpu.SemaphoreType.DMA,
        pltpu.SemaphoreType.DMA,
        pltpu.SemaphoreType.DMA,
    ],
)
def _gather_kernel(cos_hbm, sin_hbm, pos_hbm, cos_out, sin_out,
                   idx_v, cbuf, sbuf, gcsem, gssem, wcsem, wssem):
    wid = lax.axis_index("s") * NC + lax.axis_index("c")
    base = wid * BPW
    pltpu.sync_copy(pos_hbm.at[pl.ds(base, BPW)], idx_v)
    gc, gs, writes = {}, {}, []
    for c in range(NCH):
        idx = idx_v.at[pl.ds(c * CH, CH)]
        gc[c] = pltpu.async_copy(cos_hbm.at[idx], cbuf.at[c], gcsem)
        gs[c] = pltpu.async_copy(sin_hbm.at[idx], sbuf.at[c], gssem)
    for c in range(NCH):
        out = pl.ds(base + c * CH, CH)
        gc[c].wait()
        writes.append(pltpu.async_copy(cbuf.at[c], cos_out.at[out, 0], wcsem))
        writes.append(pltpu.async_copy(cbuf.at[c], cos_out.at[out, 1], wcsem))
        gs[c].wait()
        writes.append(pltpu.async_copy(sbuf.at[c], sin_out.at[out, 0], wssem))
        writes.append(pltpu.async_copy(sbuf.at[c], sin_out.at[out, 1], wssem))
    for w in writes:
        w.wait()


def kernel(cos_cached, sin_cached, position_ids):
    b, s = position_ids.shape
    cos_h = cos_cached.reshape(2 * MAXP, HALF)
    sin_h = sin_cached.reshape(2 * MAXP, HALF)
    pos = (position_ids.astype(jnp.int32) * 2).reshape(TOTAL)
    cos_full, sin_full = _gather_kernel(cos_h, sin_h, pos)
    return (cos_full.reshape(b, s, DIM), sin_full.reshape(b, s, DIM))
